# baseline (speedup 1.0000x reference)
import jax
import jax.numpy as jnp
from jax.experimental import pallas as pl
from jax.experimental.pallas import tpu as pltpu

LOW_T = 0.3
HIGH_T = 0.7
BETA = 1.0 / 9.0

N, A, H, W, G = 2, 3, 200, 200, 50
HW = H * W
M = A * HW
CHUNK = 4000
NCH = HW // CHUNK


def _iou_tile(gx1, gy1, gx2, gy2, garea, ax1, ay1, ax2, ay2, aarea):
    ltx = jnp.maximum(gx1, ax1)
    lty = jnp.maximum(gy1, ay1)
    rbx = jnp.minimum(gx2, ax2)
    rby = jnp.minimum(gy2, ay2)
    w = jnp.clip(rbx - ltx, 0.0)
    h = jnp.clip(rby - lty, 0.0)
    inter = w * h
    union = garea + aarea - inter
    return inter / union


def _rpn_loss_kernel(cls_ref, reg_ref, gt_ref, out_ref, iou_ref):
    giota = jax.lax.broadcasted_iota(jnp.int32, (G, 1), 0).astype(jnp.float32)

    cls_acc = jnp.zeros((1, 1), jnp.float32)
    reg_acc = jnp.zeros((1, 1), jnp.float32)

    for n in range(N):
        gx1 = gt_ref[n * 4 + 0]
        gy1 = gt_ref[n * 4 + 1]
        gx2 = gt_ref[n * 4 + 2]
        gy2 = gt_ref[n * 4 + 3]
        garea = (gx2 - gx1) * (gy2 - gy1)

        def anchor_boxes(a, c):
            r = (n * A + a) * 4
            sl = slice(c * CHUNK, (c + 1) * CHUNK)
            ax1 = reg_ref[r + 0 : r + 1, sl]
            ay1 = reg_ref[r + 1 : r + 2, sl]
            ax2 = reg_ref[r + 2 : r + 3, sl]
            ay2 = reg_ref[r + 3 : r + 4, sl]
            return ax1, ay1, ax2, ay2

        pergt = jnp.full((G, 1), -jnp.inf, jnp.float32)
        for a in range(A):
            for c in range(NCH):
                ax1, ay1, ax2, ay2 = anchor_boxes(a, c)
                aarea = (ax2 - ax1) * (ay2 - ay1)
                iou = _iou_tile(
                    gx1, gy1, gx2, gy2, garea, ax1, ay1, ax2, ay2, aarea
                )
                iou_ref[:, slice(a * HW + c * CHUNK, a * HW + (c + 1) * CHUNK)] = iou
                pergt = jnp.maximum(pergt, jnp.max(iou, axis=1, keepdims=True))

        for a in range(A):
            for c in range(NCH):
                ax1, ay1, ax2, ay2 = anchor_boxes(a, c)
                iou = iou_ref[:, slice(a * HW + c * CHUNK, a * HW + (c + 1) * CHUNK)]
                best = jnp.max(iou, axis=0, keepdims=True)
                idx = jnp.min(
                    jnp.where(iou == best, giota, jnp.float32(G)),
                    axis=0,
                    keepdims=True,
                )
                force = (
                    jnp.max(
                        jnp.where(iou == pergt, 1.0, 0.0), axis=0, keepdims=True
                    )
                    > 0.0
                )
                onehot_f = (giota == idx).astype(jnp.float32)
                gtmat = gt_ref[n * 4 : n * 4 + 4, :, 0]
                tmat = jax.lax.dot_general(
                    gtmat,
                    onehot_f,
                    (((1,), (0,)), ((), ())),
                    precision=jax.lax.Precision.HIGHEST,
                    preferred_element_type=jnp.float32,
                )
                tx1 = tmat[0:1, :]
                ty1 = tmat[1:2, :]
                tx2 = tmat[2:3, :]
                ty2 = tmat[3:4, :]

                pos = force | (best >= HIGH_T)
                label = jnp.where(pos, 1.0, jnp.where(best < LOW_T, 0.0, -1.0))
                tx1 = jnp.where(pos, tx1, gx1[0:1, :])
                ty1 = jnp.where(pos, ty1, gy1[0:1, :])
                tx2 = jnp.where(pos, tx2, gx2[0:1, :])
                ty2 = jnp.where(pos, ty2, gy2[0:1, :])

                rc = n * A + a
                x = cls_ref[rc : rc + 1, slice(c * CHUNK, (c + 1) * CHUNK)]
                bce = (
                    jnp.maximum(x, 0.0)
                    - x * label
                    + jnp.log1p(jnp.exp(-jnp.abs(x)))
                )
                cls_acc = cls_acc + jnp.sum(bce, keepdims=True)

                for av, tv in ((ax1, tx1), (ay1, ty1), (ax2, tx2), (ay2, ty2)):
                    d = jnp.abs(av - tv)
                    sl1 = jnp.where(d < BETA, 0.5 * d * d / BETA, d - 0.5 * BETA)
                    reg_acc = reg_acc + jnp.sum(sl1, keepdims=True)

    total = cls_acc / jnp.float32(N * M) + reg_acc / jnp.float32(N * M * 4)
    out_ref[...] = total


def kernel(cls_level0, reg_level0, gt_boxes, gt_labels):
    del gt_labels
    cls2 = cls_level0.reshape(N * A, HW)
    reg2 = reg_level0.reshape(N * A * 4, HW)
    gt3 = jnp.transpose(gt_boxes, (0, 2, 1)).reshape(N * 4, G, 1)
    out = pl.pallas_call(
        _rpn_loss_kernel,
        out_shape=jax.ShapeDtypeStruct((1, 1), jnp.float32),
        scratch_shapes=[pltpu.VMEM((G, M), jnp.float32)],
    )(cls2, reg2, gt3)
    return out[0, 0]


# speedup vs baseline: 1.0606x; 1.0606x over previous
import jax
import jax.numpy as jnp
from jax.experimental import pallas as pl
from jax.experimental.pallas import tpu as pltpu

LOW_T = 0.3
HIGH_T = 0.7
BETA = 1.0 / 9.0

N, A, H, W, G = 2, 3, 200, 200, 50
HW = H * W
M = A * HW
CHUNK = 20000
NCH = HW // CHUNK


def _iou_tile(gx1, gy1, gx2, gy2, garea, ax1, ay1, ax2, ay2, aarea):
    ltx = jnp.maximum(gx1, ax1)
    lty = jnp.maximum(gy1, ay1)
    rbx = jnp.minimum(gx2, ax2)
    rby = jnp.minimum(gy2, ay2)
    w = jnp.clip(rbx - ltx, 0.0)
    h = jnp.clip(rby - lty, 0.0)
    inter = w * h
    union = garea + aarea - inter
    return inter / union


def _rpn_loss_kernel(cls_ref, reg_ref, gt_ref, out_ref, iou_ref):
    giota = jax.lax.broadcasted_iota(jnp.int32, (G, 1), 0).astype(jnp.float32)

    cls_acc = jnp.zeros((1, 1), jnp.float32)
    reg_acc = jnp.zeros((1, 1), jnp.float32)

    for n in range(N):
        gx1 = gt_ref[n * 4 + 0]
        gy1 = gt_ref[n * 4 + 1]
        gx2 = gt_ref[n * 4 + 2]
        gy2 = gt_ref[n * 4 + 3]
        garea = (gx2 - gx1) * (gy2 - gy1)

        def anchor_boxes(a, c):
            r = (n * A + a) * 4
            sl = slice(c * CHUNK, (c + 1) * CHUNK)
            ax1 = reg_ref[r + 0 : r + 1, sl]
            ay1 = reg_ref[r + 1 : r + 2, sl]
            ax2 = reg_ref[r + 2 : r + 3, sl]
            ay2 = reg_ref[r + 3 : r + 4, sl]
            return ax1, ay1, ax2, ay2

        pergt = jnp.full((G, 1), -jnp.inf, jnp.float32)
        for a in range(A):
            for c in range(NCH):
                ax1, ay1, ax2, ay2 = anchor_boxes(a, c)
                aarea = (ax2 - ax1) * (ay2 - ay1)
                iou = _iou_tile(
                    gx1, gy1, gx2, gy2, garea, ax1, ay1, ax2, ay2, aarea
                )
                iou_ref[:, slice(a * HW + c * CHUNK, a * HW + (c + 1) * CHUNK)] = iou
                pergt = jnp.maximum(pergt, jnp.max(iou, axis=1, keepdims=True))

        for a in range(A):
            for c in range(NCH):
                ax1, ay1, ax2, ay2 = anchor_boxes(a, c)
                iou = iou_ref[:, slice(a * HW + c * CHUNK, a * HW + (c + 1) * CHUNK)]
                best = jnp.max(iou, axis=0, keepdims=True)
                idx = jnp.min(
                    jnp.where(iou == best, giota, jnp.float32(G)),
                    axis=0,
                    keepdims=True,
                )
                force = (
                    jnp.max(
                        jnp.where(iou == pergt, 1.0, 0.0), axis=0, keepdims=True
                    )
                    > 0.0
                )
                onehot_f = (giota == idx).astype(jnp.float32)
                gtmat = gt_ref[n * 4 : n * 4 + 4, :, 0]
                tmat = jax.lax.dot_general(
                    gtmat,
                    onehot_f,
                    (((1,), (0,)), ((), ())),
                    precision=jax.lax.Precision.HIGHEST,
                    preferred_element_type=jnp.float32,
                )
                tx1 = tmat[0:1, :]
                ty1 = tmat[1:2, :]
                tx2 = tmat[2:3, :]
                ty2 = tmat[3:4, :]

                pos = force | (best >= HIGH_T)
                label = jnp.where(pos, 1.0, jnp.where(best < LOW_T, 0.0, -1.0))
                tx1 = jnp.where(pos, tx1, gx1[0:1, :])
                ty1 = jnp.where(pos, ty1, gy1[0:1, :])
                tx2 = jnp.where(pos, tx2, gx2[0:1, :])
                ty2 = jnp.where(pos, ty2, gy2[0:1, :])

                rc = n * A + a
                x = cls_ref[rc : rc + 1, slice(c * CHUNK, (c + 1) * CHUNK)]
                bce = (
                    jnp.maximum(x, 0.0)
                    - x * label
                    + jnp.log1p(jnp.exp(-jnp.abs(x)))
                )
                cls_acc = cls_acc + jnp.sum(bce, keepdims=True)

                for av, tv in ((ax1, tx1), (ay1, ty1), (ax2, tx2), (ay2, ty2)):
                    d = jnp.abs(av - tv)
                    sl1 = jnp.where(d < BETA, 0.5 * d * d / BETA, d - 0.5 * BETA)
                    reg_acc = reg_acc + jnp.sum(sl1, keepdims=True)

    total = cls_acc / jnp.float32(N * M) + reg_acc / jnp.float32(N * M * 4)
    out_ref[...] = total


def kernel(cls_level0, reg_level0, gt_boxes, gt_labels):
    del gt_labels
    cls2 = cls_level0.reshape(N * A, HW)
    reg2 = reg_level0.reshape(N * A * 4, HW)
    gt3 = jnp.transpose(gt_boxes, (0, 2, 1)).reshape(N * 4, G, 1)
    out = pl.pallas_call(
        _rpn_loss_kernel,
        out_shape=jax.ShapeDtypeStruct((1, 1), jnp.float32),
        scratch_shapes=[pltpu.VMEM((G, M), jnp.float32)],
    )(cls2, reg2, gt3)
    return out[0, 0]


# speedup vs baseline: 1.0940x; 1.0315x over previous
import jax
import jax.numpy as jnp
from jax.experimental import pallas as pl
from jax.experimental.pallas import tpu as pltpu

LOW_T = 0.3
HIGH_T = 0.7
BETA = 1.0 / 9.0

N, A, H, W, G = 2, 3, 200, 200, 50
HW = H * W
M = A * HW
CHUNK = 8000
NCH = HW // CHUNK


def _iou_tile(gx1, gy1, gx2, gy2, garea, ax1, ay1, ax2, ay2, aarea):
    ltx = jnp.maximum(gx1, ax1)
    lty = jnp.maximum(gy1, ay1)
    rbx = jnp.minimum(gx2, ax2)
    rby = jnp.minimum(gy2, ay2)
    w = jnp.clip(rbx - ltx, 0.0)
    h = jnp.clip(rby - lty, 0.0)
    inter = w * h
    union = garea + aarea - inter
    return inter / union


def _rpn_loss_kernel(cls_ref, reg_ref, gt_ref, out_ref, iou_ref):
    giota = jax.lax.broadcasted_iota(jnp.int32, (G, 1), 0).astype(jnp.float32)

    cls_acc = jnp.zeros((1, 1), jnp.float32)
    reg_acc = jnp.zeros((1, 1), jnp.float32)

    for n in range(N):
        gx1 = gt_ref[n * 4 + 0]
        gy1 = gt_ref[n * 4 + 1]
        gx2 = gt_ref[n * 4 + 2]
        gy2 = gt_ref[n * 4 + 3]
        garea = (gx2 - gx1) * (gy2 - gy1)

        def anchor_boxes(a, c):
            r = (n * A + a) * 4
            sl = slice(c * CHUNK, (c + 1) * CHUNK)
            ax1 = reg_ref[r + 0 : r + 1, sl]
            ay1 = reg_ref[r + 1 : r + 2, sl]
            ax2 = reg_ref[r + 2 : r + 3, sl]
            ay2 = reg_ref[r + 3 : r + 4, sl]
            return ax1, ay1, ax2, ay2

        pergt = jnp.full((G, 1), -jnp.inf, jnp.float32)
        for a in range(A):
            for c in range(NCH):
                ax1, ay1, ax2, ay2 = anchor_boxes(a, c)
                aarea = (ax2 - ax1) * (ay2 - ay1)
                iou = _iou_tile(
                    gx1, gy1, gx2, gy2, garea, ax1, ay1, ax2, ay2, aarea
                )
                iou_ref[:, slice(a * HW + c * CHUNK, a * HW + (c + 1) * CHUNK)] = iou
                pergt = jnp.maximum(pergt, jnp.max(iou, axis=1, keepdims=True))

        for a in range(A):
            for c in range(NCH):
                ax1, ay1, ax2, ay2 = anchor_boxes(a, c)
                iou = iou_ref[:, slice(a * HW + c * CHUNK, a * HW + (c + 1) * CHUNK)]
                best = jnp.max(iou, axis=0, keepdims=True)
                idx = jnp.min(
                    jnp.where(iou == best, giota, jnp.float32(G)),
                    axis=0,
                    keepdims=True,
                )
                force = (
                    jnp.max(
                        jnp.where(iou == pergt, 1.0, 0.0), axis=0, keepdims=True
                    )
                    > 0.0
                )
                onehot_f = (giota == idx).astype(jnp.float32)
                gtmat = gt_ref[n * 4 : n * 4 + 4, :, 0]
                tmat = jax.lax.dot_general(
                    gtmat,
                    onehot_f,
                    (((1,), (0,)), ((), ())),
                    precision=jax.lax.Precision.HIGHEST,
                    preferred_element_type=jnp.float32,
                )
                tx1 = tmat[0:1, :]
                ty1 = tmat[1:2, :]
                tx2 = tmat[2:3, :]
                ty2 = tmat[3:4, :]

                pos = force | (best >= HIGH_T)
                label = jnp.where(pos, 1.0, jnp.where(best < LOW_T, 0.0, -1.0))
                tx1 = jnp.where(pos, tx1, gx1[0:1, :])
                ty1 = jnp.where(pos, ty1, gy1[0:1, :])
                tx2 = jnp.where(pos, tx2, gx2[0:1, :])
                ty2 = jnp.where(pos, ty2, gy2[0:1, :])

                rc = n * A + a
                x = cls_ref[rc : rc + 1, slice(c * CHUNK, (c + 1) * CHUNK)]
                bce = (
                    jnp.maximum(x, 0.0)
                    - x * label
                    + jnp.log1p(jnp.exp(-jnp.abs(x)))
                )
                cls_acc = cls_acc + jnp.sum(bce, keepdims=True)

                for av, tv in ((ax1, tx1), (ay1, ty1), (ax2, tx2), (ay2, ty2)):
                    d = jnp.abs(av - tv)
                    sl1 = jnp.where(d < BETA, 0.5 * d * d / BETA, d - 0.5 * BETA)
                    reg_acc = reg_acc + jnp.sum(sl1, keepdims=True)

    total = cls_acc / jnp.float32(N * M) + reg_acc / jnp.float32(N * M * 4)
    out_ref[...] = total


def kernel(cls_level0, reg_level0, gt_boxes, gt_labels):
    del gt_labels
    cls2 = cls_level0.reshape(N * A, HW)
    reg2 = reg_level0.reshape(N * A * 4, HW)
    gt3 = jnp.transpose(gt_boxes, (0, 2, 1)).reshape(N * 4, G, 1)
    out = pl.pallas_call(
        _rpn_loss_kernel,
        out_shape=jax.ShapeDtypeStruct((1, 1), jnp.float32),
        scratch_shapes=[pltpu.VMEM((G, M), jnp.float32)],
    )(cls2, reg2, gt3)
    return out[0, 0]


# speedup vs baseline: 1.1080x; 1.0128x over previous
import jax
import jax.numpy as jnp
from jax.experimental import pallas as pl
from jax.experimental.pallas import tpu as pltpu

LOW_T = 0.3
HIGH_T = 0.7
BETA = 1.0 / 9.0

N, A, H, W, G = 2, 3, 200, 200, 50
HW = H * W
M = A * HW
CHUNK = 8000
NCH = HW // CHUNK


def _iou_tile(gx1, gy1, gx2, gy2, garea, ax1, ay1, ax2, ay2, aarea):
    ltx = jnp.maximum(gx1, ax1)
    lty = jnp.maximum(gy1, ay1)
    rbx = jnp.minimum(gx2, ax2)
    rby = jnp.minimum(gy2, ay2)
    w = jnp.clip(rbx - ltx, 0.0)
    h = jnp.clip(rby - lty, 0.0)
    inter = w * h
    union = garea + aarea - inter
    return inter / union


def _rpn_loss_kernel(cls_ref, reg_ref, gt_ref, out_ref, iou_ref):
    giota = jax.lax.broadcasted_iota(jnp.int32, (G, 1), 0).astype(jnp.float32)

    cls_acc = jnp.zeros((1, 1), jnp.float32)
    reg_acc = jnp.zeros((1, 1), jnp.float32)

    for n in range(N):
        gx1 = gt_ref[n * 4 + 0]
        gy1 = gt_ref[n * 4 + 1]
        gx2 = gt_ref[n * 4 + 2]
        gy2 = gt_ref[n * 4 + 3]
        garea = (gx2 - gx1) * (gy2 - gy1)

        def anchor_boxes(a, c):
            r = (n * A + a) * 4
            sl = slice(c * CHUNK, (c + 1) * CHUNK)
            ax1 = reg_ref[r + 0 : r + 1, sl]
            ay1 = reg_ref[r + 1 : r + 2, sl]
            ax2 = reg_ref[r + 2 : r + 3, sl]
            ay2 = reg_ref[r + 3 : r + 4, sl]
            return ax1, ay1, ax2, ay2

        pergt = jnp.full((G, 1), -jnp.inf, jnp.float32)
        for a in range(A):
            for c in range(NCH):
                ax1, ay1, ax2, ay2 = anchor_boxes(a, c)
                aarea = (ax2 - ax1) * (ay2 - ay1)
                iou = _iou_tile(
                    gx1, gy1, gx2, gy2, garea, ax1, ay1, ax2, ay2, aarea
                )
                iou_ref[:, slice(a * HW + c * CHUNK, a * HW + (c + 1) * CHUNK)] = iou
                pergt = jnp.maximum(pergt, jnp.max(iou, axis=1, keepdims=True))

        for a in range(A):
            for c in range(NCH):
                ax1, ay1, ax2, ay2 = anchor_boxes(a, c)
                iou = iou_ref[:, slice(a * HW + c * CHUNK, a * HW + (c + 1) * CHUNK)]
                best = jnp.max(iou, axis=0, keepdims=True)
                idx = jnp.min(
                    jnp.where(iou == best, giota, jnp.float32(G)),
                    axis=0,
                    keepdims=True,
                )
                force = (
                    jnp.max(iou - pergt, axis=0, keepdims=True) == 0.0
                )
                onehot_f = (giota == idx).astype(jnp.float32)
                gtmat = gt_ref[n * 4 : n * 4 + 4, :, 0]
                tmat = jax.lax.dot_general(
                    gtmat,
                    onehot_f,
                    (((1,), (0,)), ((), ())),
                    precision=jax.lax.Precision.HIGHEST,
                    preferred_element_type=jnp.float32,
                )
                tx1 = tmat[0:1, :]
                ty1 = tmat[1:2, :]
                tx2 = tmat[2:3, :]
                ty2 = tmat[3:4, :]

                pos = force | (best >= HIGH_T)
                label = jnp.where(pos, 1.0, jnp.where(best < LOW_T, 0.0, -1.0))
                tx1 = jnp.where(pos, tx1, gx1[0:1, :])
                ty1 = jnp.where(pos, ty1, gy1[0:1, :])
                tx2 = jnp.where(pos, tx2, gx2[0:1, :])
                ty2 = jnp.where(pos, ty2, gy2[0:1, :])

                rc = n * A + a
                x = cls_ref[rc : rc + 1, slice(c * CHUNK, (c + 1) * CHUNK)]
                bce = (
                    jnp.maximum(x, 0.0)
                    - x * label
                    + jnp.log1p(jnp.exp(-jnp.abs(x)))
                )
                cls_acc = cls_acc + jnp.sum(bce, keepdims=True)

                for av, tv in ((ax1, tx1), (ay1, ty1), (ax2, tx2), (ay2, ty2)):
                    d = jnp.abs(av - tv)
                    sl1 = jnp.where(d < BETA, 0.5 * d * d / BETA, d - 0.5 * BETA)
                    reg_acc = reg_acc + jnp.sum(sl1, keepdims=True)

    total = cls_acc / jnp.float32(N * M) + reg_acc / jnp.float32(N * M * 4)
    out_ref[...] = total


def kernel(cls_level0, reg_level0, gt_boxes, gt_labels):
    del gt_labels
    cls2 = cls_level0.reshape(N * A, HW)
    reg2 = reg_level0.reshape(N * A * 4, HW)
    gt3 = jnp.transpose(gt_boxes, (0, 2, 1)).reshape(N * 4, G, 1)
    out = pl.pallas_call(
        _rpn_loss_kernel,
        out_shape=jax.ShapeDtypeStruct((1, 1), jnp.float32),
        scratch_shapes=[pltpu.VMEM((G, M), jnp.float32)],
    )(cls2, reg2, gt3)
    return out[0, 0]


# speedup vs baseline: 1.1089x; 1.0008x over previous
import jax
import jax.numpy as jnp
from jax.experimental import pallas as pl
from jax.experimental.pallas import tpu as pltpu

LOW_T = 0.3
HIGH_T = 0.7
BETA = 1.0 / 9.0

N, A, H, W, G = 2, 3, 200, 200, 50
HW = H * W
M = A * HW
CHUNK = 8000
NCH = HW // CHUNK


def _iou_tile(gx1, gy1, gx2, gy2, garea, ax1, ay1, ax2, ay2, aarea):
    ltx = jnp.maximum(gx1, ax1)
    lty = jnp.maximum(gy1, ay1)
    rbx = jnp.minimum(gx2, ax2)
    rby = jnp.minimum(gy2, ay2)
    w = jnp.clip(rbx - ltx, 0.0)
    h = jnp.clip(rby - lty, 0.0)
    inter = w * h
    union = garea + aarea - inter
    return inter / union


def _rpn_loss_kernel(cls_ref, reg_ref, gt_ref, out_ref, iou_ref):
    giota = jax.lax.broadcasted_iota(jnp.int32, (G, 1), 0).astype(jnp.float32)

    cls_acc = jnp.zeros((1, 1), jnp.float32)
    reg_acc = jnp.zeros((1, 1), jnp.float32)

    for n in range(N):
        gx1 = gt_ref[n * 4 + 0]
        gy1 = gt_ref[n * 4 + 1]
        gx2 = gt_ref[n * 4 + 2]
        gy2 = gt_ref[n * 4 + 3]
        garea = (gx2 - gx1) * (gy2 - gy1)

        def anchor_boxes(a, c):
            r = (n * A + a) * 4
            sl = slice(c * CHUNK, (c + 1) * CHUNK)
            ax1 = reg_ref[r + 0 : r + 1, sl]
            ay1 = reg_ref[r + 1 : r + 2, sl]
            ax2 = reg_ref[r + 2 : r + 3, sl]
            ay2 = reg_ref[r + 3 : r + 4, sl]
            return ax1, ay1, ax2, ay2

        chunk_maxes = []
        for a in range(A):
            for c in range(NCH):
                ax1, ay1, ax2, ay2 = anchor_boxes(a, c)
                aarea = (ax2 - ax1) * (ay2 - ay1)
                iou = _iou_tile(
                    gx1, gy1, gx2, gy2, garea, ax1, ay1, ax2, ay2, aarea
                )
                iou_ref[:, slice(a * HW + c * CHUNK, a * HW + (c + 1) * CHUNK)] = iou
                chunk_maxes.append(jnp.max(iou, axis=1, keepdims=True))
        while len(chunk_maxes) > 1:
            chunk_maxes = [
                jnp.maximum(*chunk_maxes[i : i + 2])
                if i + 1 < len(chunk_maxes)
                else chunk_maxes[i]
                for i in range(0, len(chunk_maxes), 2)
            ]
        pergt = chunk_maxes[0]

        for a in range(A):
            for c in range(NCH):
                ax1, ay1, ax2, ay2 = anchor_boxes(a, c)
                iou = iou_ref[:, slice(a * HW + c * CHUNK, a * HW + (c + 1) * CHUNK)]
                best = jnp.max(iou, axis=0, keepdims=True)
                idx = jnp.min(
                    jnp.where(iou == best, giota, jnp.float32(G)),
                    axis=0,
                    keepdims=True,
                )
                force = (
                    jnp.max(iou - pergt, axis=0, keepdims=True) == 0.0
                )
                onehot_f = (giota == idx).astype(jnp.float32)
                gtmat = gt_ref[n * 4 : n * 4 + 4, :, 0]
                tmat = jax.lax.dot_general(
                    gtmat,
                    onehot_f,
                    (((1,), (0,)), ((), ())),
                    precision=jax.lax.Precision.HIGHEST,
                    preferred_element_type=jnp.float32,
                )
                tx1 = tmat[0:1, :]
                ty1 = tmat[1:2, :]
                tx2 = tmat[2:3, :]
                ty2 = tmat[3:4, :]

                pos = force | (best >= HIGH_T)
                label = jnp.where(pos, 1.0, jnp.where(best < LOW_T, 0.0, -1.0))
                tx1 = jnp.where(pos, tx1, gx1[0:1, :])
                ty1 = jnp.where(pos, ty1, gy1[0:1, :])
                tx2 = jnp.where(pos, tx2, gx2[0:1, :])
                ty2 = jnp.where(pos, ty2, gy2[0:1, :])

                rc = n * A + a
                x = cls_ref[rc : rc + 1, slice(c * CHUNK, (c + 1) * CHUNK)]
                bce = (
                    jnp.maximum(x, 0.0)
                    - x * label
                    + jnp.log1p(jnp.exp(-jnp.abs(x)))
                )
                cls_acc = cls_acc + jnp.sum(bce, keepdims=True)

                for av, tv in ((ax1, tx1), (ay1, ty1), (ax2, tx2), (ay2, ty2)):
                    d = jnp.abs(av - tv)
                    sl1 = jnp.where(d < BETA, 0.5 * d * d / BETA, d - 0.5 * BETA)
                    reg_acc = reg_acc + jnp.sum(sl1, keepdims=True)

    total = cls_acc / jnp.float32(N * M) + reg_acc / jnp.float32(N * M * 4)
    out_ref[...] = total


def kernel(cls_level0, reg_level0, gt_boxes, gt_labels):
    del gt_labels
    cls2 = cls_level0.reshape(N * A, HW)
    reg2 = reg_level0.reshape(N * A * 4, HW)
    gt3 = jnp.transpose(gt_boxes, (0, 2, 1)).reshape(N * 4, G, 1)
    out = pl.pallas_call(
        _rpn_loss_kernel,
        out_shape=jax.ShapeDtypeStruct((1, 1), jnp.float32),
        scratch_shapes=[pltpu.VMEM((G, M), jnp.float32)],
    )(cls2, reg2, gt3)
    return out[0, 0]
